# baseline (speedup 1.0000x reference)
import functools

import jax
import jax.numpy as jnp
from jax import lax
from jax.experimental import pallas as pl
from jax.experimental.pallas import tpu as pltpu
from jax.experimental.pallas import tpu_sc as plsc

NC = 2
NS = 16
LANES = 16
EC = 128


def _deg_body(nch, npad, dst_hbm, ew_hbm, out_hbm, dst_v, ew_v, acc_v):
    c = lax.axis_index("c")
    s = lax.axis_index("s")
    pltpu.sync_copy(dst_hbm.at[c, s], dst_v)
    pltpu.sync_copy(ew_hbm.at[c, s], ew_v)

    zero16 = jnp.zeros((LANES,), jnp.float32)

    def zr(i, _):
        acc_v[pl.ds(i * LANES, LANES)] = zero16
        return 0

    lax.fori_loop(0, npad // LANES, zr, 0)

    def chunk(ci, _):
        def grp(j, _):
            sl = pl.ds(j * LANES, LANES)
            plsc.addupdate_scatter(acc_v, [dst_v[ci, sl]], ew_v[ci, sl])
            return 0

        lax.fori_loop(0, EC // LANES, grp, 0)
        return 0

    lax.fori_loop(0, nch, chunk, 0)
    w = c * NS + s
    pltpu.sync_copy(acc_v, out_hbm.at[w])


def _deg_call(dst_r, ew_r, nch, npad):
    mesh = plsc.VectorSubcoreMesh(core_axis_name="c", subcore_axis_name="s")
    return pl.kernel(
        functools.partial(_deg_body, nch, npad),
        out_type=jax.ShapeDtypeStruct((NC * NS, npad), jnp.float32),
        mesh=mesh,
        compiler_params=pltpu.CompilerParams(needs_layout_passes=False),
        scratch_types=[
            pltpu.VMEM((nch, EC), jnp.int32),
            pltpu.VMEM((nch, EC), jnp.float32),
            pltpu.VMEM((npad,), jnp.float32),
        ],
    )(dst_r, ew_r)


def _norm_body(nch, npad, src_hbm, dst_hbm, ew_hbm, dis_hbm, out_hbm,
               src_v, dst_v, ew_v, dis_v, norm_v):
    c = lax.axis_index("c")
    s = lax.axis_index("s")
    pltpu.sync_copy(src_hbm.at[c, s], src_v)
    pltpu.sync_copy(dst_hbm.at[c, s], dst_v)
    pltpu.sync_copy(ew_hbm.at[c, s], ew_v)
    pltpu.sync_copy(dis_hbm, dis_v)

    def chunk(ci, _):
        def grp(j, _):
            sl = pl.ds(j * LANES, LANES)
            norm_v[ci, sl] = (plsc.load_gather(dis_v, [src_v[ci, sl]])
                              * ew_v[ci, sl]
                              * plsc.load_gather(dis_v, [dst_v[ci, sl]]))
            return 0

        lax.fori_loop(0, EC // LANES, grp, 0)
        return 0

    lax.fori_loop(0, nch, chunk, 0)
    pltpu.sync_copy(norm_v, out_hbm.at[c, s])


def _norm_call(src_r, dst_r, ew_r, dis, nch, npad):
    mesh = plsc.VectorSubcoreMesh(core_axis_name="c", subcore_axis_name="s")
    return pl.kernel(
        functools.partial(_norm_body, nch, npad),
        out_type=jax.ShapeDtypeStruct((NC, NS, nch, EC), jnp.float32),
        mesh=mesh,
        compiler_params=pltpu.CompilerParams(needs_layout_passes=False),
        scratch_types=[
            pltpu.VMEM((nch, EC), jnp.int32),
            pltpu.VMEM((nch, EC), jnp.int32),
            pltpu.VMEM((nch, EC), jnp.float32),
            pltpu.VMEM((npad,), jnp.float32),
            pltpu.VMEM((nch, EC), jnp.float32),
        ],
    )(src_r, dst_r, ew_r, dis)


ECA = 64
NBUF = 3


def _agg_body(nch0, nch1, npad, hd, tab_hbm, packed_hbm, norm_hbm, out_hbm,
              packed_v, norm_v, srcb, dstb, row0, row1, row2,
              acc_sh, sg0, sg1, sg2, ss0, ss1, ss2, sz):
    c = lax.axis_index("c")
    s = lax.axis_index("s")
    nch2 = jnp.where(c == 0, 2 * nch0, 2 * nch1)
    start = jnp.where(c == 0, s * nch0, NS * nch0 + s * nch1)
    rows = (row0, row1, row2)
    sgs = (sg0, sg1, sg2)
    sss = (ss0, ss1, ss2)
    pltpu.sync_copy(packed_hbm.at[pl.ds(start, max(nch0, nch1))], packed_v)
    pltpu.sync_copy(norm_hbm.at[pl.ds(start, max(nch0, nch1))], norm_v)

    zero16 = jnp.zeros((LANES,), jnp.float32)
    nj = hd // LANES

    def zrow(e, _):
        for j in range(nj):
            row2[e, pl.ds(j * LANES, LANES)] = zero16
        return 0

    lax.fori_loop(0, ECA, zrow, 0)

    rows_per_tile = npad // NS
    base = s * rows_per_tile
    nfull, rem = rows_per_tile // ECA, rows_per_tile % ECA
    zcopies = []
    for k in range(nfull):
        zcopies.append(pltpu.async_copy(
            row2, acc_sh.at[pl.ds(base + k * ECA, ECA)], sz))
    if rem:
        zcopies.append(pltpu.async_copy(
            row2.at[pl.ds(0, rem)],
            acc_sh.at[pl.ds(base + nfull * ECA, rem)], sz))

    def prep(prow, pcol, b):
        for q in range(ECA // LANES):
            sl = pl.ds(pcol * ECA + q * LANES, LANES)
            pv = packed_v[prow, sl]
            osl = pl.ds(q * LANES, LANES)
            srcb[b, osl] = lax.shift_right_logical(pv, 16)
            dstb[b, osl] = lax.bitwise_and(pv, 0xFFFF)
        pltpu.async_copy(tab_hbm.at[srcb.at[b]], rows[b], sgs[b])

    prep(0, 0, 0)
    prep(0, 1, 1)
    for zc in zcopies:
        zc.wait()
    plsc.subcore_barrier()

    def step(k, _):
        for u in range(6):
            ci = 6 * k + u
            b = u % 3
            r = 3 * k + u // 2
            col = u % 2
            pltpu.make_async_copy(tab_hbm.at[srcb.at[b]], rows[b],
                                  sgs[b]).wait()

            def sgrp(g, _):
                nv = norm_v[r, pl.ds(col * ECA + g * LANES, LANES)]
                for lane in range(LANES):
                    sc = jnp.full((LANES,), nv[lane], jnp.float32)
                    e = g * LANES + lane
                    for j in range(nj):
                        fsl = pl.ds(j * LANES, LANES)
                        rows[b][e, fsl] = rows[b][e, fsl] * sc
                return 0

            lax.fori_loop(0, ECA // LANES, sgrp, 0)
            pltpu.async_copy(rows[b], acc_sh.at[dstb.at[b]], sss[b],
                             add=True)

            bp = (b + 2) % 3

            @pl.when(ci >= 1)
            def _():
                pltpu.make_async_copy(rows[bp], acc_sh.at[dstb.at[bp]],
                                      sss[bp]).wait()

            @pl.when(ci + 2 < nch2)
            def _():
                prep(3 * k + (u + 2) // 2, (u + 2) % 2, bp)
        return 0

    lax.fori_loop(0, nch2 // 6, step, 0)
    pltpu.make_async_copy(rows[2], acc_sh.at[dstb.at[2]], sss[2]).wait()
    plsc.subcore_barrier()

    ocopies = []
    for k in range(nfull):
        ocopies.append(pltpu.async_copy(
            acc_sh.at[pl.ds(base + k * ECA, ECA)],
            out_hbm.at[c, pl.ds(base + k * ECA, ECA)], sz))
    if rem:
        ocopies.append(pltpu.async_copy(
            acc_sh.at[pl.ds(base + nfull * ECA, rem)],
            out_hbm.at[c, pl.ds(base + nfull * ECA, rem)], sz))
    for oc in ocopies:
        oc.wait()


def _agg_call(tab, packed_r, norm_r, nch0, nch1, npad, hd):
    mesh = plsc.VectorSubcoreMesh(core_axis_name="c", subcore_axis_name="s")
    nmax = max(nch0, nch1)
    return pl.kernel(
        functools.partial(_agg_body, nch0, nch1, npad, hd),
        out_type=jax.ShapeDtypeStruct((NC, npad, hd), jnp.float32),
        mesh=mesh,
        compiler_params=pltpu.CompilerParams(needs_layout_passes=False,
                                             use_tc_tiling_on_sc=False),
        scratch_types=[
            pltpu.VMEM((nmax, EC), jnp.int32),
            pltpu.VMEM((nmax, EC), jnp.float32),
            pltpu.VMEM((NBUF, ECA), jnp.int32),
            pltpu.VMEM((NBUF, ECA), jnp.int32),
            pltpu.VMEM((ECA, hd), jnp.float32),
            pltpu.VMEM((ECA, hd), jnp.float32),
            pltpu.VMEM((ECA, hd), jnp.float32),
            pltpu.MemorySpace.VMEM_SHARED((npad, hd), jnp.float32),
            pltpu.SemaphoreType.DMA,
            pltpu.SemaphoreType.DMA,
            pltpu.SemaphoreType.DMA,
            pltpu.SemaphoreType.DMA,
            pltpu.SemaphoreType.DMA,
            pltpu.SemaphoreType.DMA,
            pltpu.SemaphoreType.DMA,
        ],
    )(tab, packed_r, norm_r)


def _tca_body(x_ref, w_ref, degp_ref, g_ref, dis_ref):
    g_ref[...] = jnp.dot(x_ref[...], w_ref[...],
                         preferred_element_type=jnp.float32)

    @pl.when(pl.program_id(0) == 0)
    def _():
        deg = jnp.sum(degp_ref[...], axis=0)
        safe = jnp.where(deg > 0, deg, 1.0)
        dis_ref[...] = jnp.where(deg > 0, lax.rsqrt(safe), 0.0)


def _tca_call(x, w, degp3, n, blk):
    npad = degp3.shape[1] * degp3.shape[2]
    grid = (n // blk,)
    return pl.pallas_call(
        _tca_body,
        grid=grid,
        in_specs=[
            pl.BlockSpec((blk, x.shape[1]), lambda i: (i, 0)),
            pl.BlockSpec(w.shape, lambda i: (0, 0)),
            pl.BlockSpec(degp3.shape, lambda i: (0, 0, 0)),
        ],
        out_specs=[
            pl.BlockSpec((blk, w.shape[1]), lambda i: (i, 0)),
            pl.BlockSpec(degp3.shape[1:], lambda i: (0, 0)),
        ],
        out_shape=[
            jax.ShapeDtypeStruct((n, w.shape[1]), jnp.float32),
            jax.ShapeDtypeStruct(degp3.shape[1:], jnp.float32),
        ],
    )(x, w, degp3)


def _tcb_body(p_ref, b_ref, w_ref, g_ref):
    h = jax.nn.relu(p_ref[0] + p_ref[1] + b_ref[0, :])
    g_ref[...] = jnp.dot(h, w_ref[...], preferred_element_type=jnp.float32)


def _tcb_call(p, b2d, w, n, blk):
    hd_in = p.shape[2]
    return pl.pallas_call(
        _tcb_body,
        grid=(n // blk,),
        in_specs=[
            pl.BlockSpec((NC, blk, hd_in), lambda i: (0, i, 0)),
            pl.BlockSpec(b2d.shape, lambda i: (0, 0)),
            pl.BlockSpec(w.shape, lambda i: (0, 0)),
        ],
        out_specs=pl.BlockSpec((blk, w.shape[1]), lambda i: (i, 0)),
        out_shape=jax.ShapeDtypeStruct((n, w.shape[1]), jnp.float32),
    )(p, b2d, w)


def _tcc_body(nclass, p_ref, b_ref, o_ref):
    t = jax.nn.relu(p_ref[0] + p_ref[1] + b_ref[0, :])
    col = lax.broadcasted_iota(jnp.int32, t.shape, 1)
    tm = jnp.where(col < nclass, t, -jnp.inf)
    m = jnp.max(tm, axis=-1, keepdims=True)
    lse = m + jnp.log(jnp.sum(jnp.exp(tm - m), axis=-1, keepdims=True))
    o_ref[...] = t - lse


def _tcc_call(p, b2d, n, blk, nclass):
    cp = p.shape[2]
    return pl.pallas_call(
        functools.partial(_tcc_body, nclass),
        grid=(n // blk,),
        in_specs=[
            pl.BlockSpec((NC, blk, cp), lambda i: (0, i, 0)),
            pl.BlockSpec(b2d.shape, lambda i: (0, 0)),
        ],
        out_specs=pl.BlockSpec((blk, cp), lambda i: (i, 0)),
        out_shape=jax.ShapeDtypeStruct((n, cp), jnp.float32),
    )(p, b2d)


def kernel(x, edge_index, edge_attr, W1, b1, W2, b2, W3, b3):
    n, d = x.shape
    h = W1.shape[1]
    c_out = W3.shape[1]
    cp = 64
    e = edge_index.shape[1]
    e_f = e + n
    nch = -(-e_f // (NC * NS * EC))
    e_pad = NC * NS * nch * EC
    npad = -(-n // 128) * 128
    blk = npad // 8

    loop_idx = jnp.arange(n, dtype=edge_index.dtype)
    src_f = jnp.concatenate([edge_index[0], loop_idx,
                             jnp.zeros((e_pad - e_f,), edge_index.dtype)])
    dst_f = jnp.concatenate([edge_index[1], loop_idx,
                             jnp.zeros((e_pad - e_f,), edge_index.dtype)])
    ew_f = jnp.concatenate([edge_attr, jnp.ones((n,), jnp.float32),
                            jnp.zeros((e_pad - e_f,), jnp.float32)])
    src_r = src_f.reshape(NC, NS, nch, EC)
    dst_r = dst_f.reshape(NC, NS, nch, EC)
    ew_r = ew_f.reshape(NC, NS, nch, EC)

    xp = jnp.pad(x, ((0, npad - n), (0, 0)))
    degp = _deg_call(dst_r, ew_r, nch, npad)
    g1, dis2 = _tca_call(xp, W1, degp.reshape(NC * NS, npad // 128, 128),
                         npad, blk)
    dis = dis2.reshape(npad)
    norm_r = _norm_call(src_r, dst_r, ew_r, dis, nch, npad)

    packed_r = (src_f * 65536 + dst_f).reshape(NC, NS, nch, EC)

    nch0 = ((2 * nch) * 13 // 32 + 3) // 6 * 6
    nch1 = 2 * nch - nch0
    packed_f = packed_r.reshape(NC * NS * nch, EC)
    norm_f = norm_r.reshape(NC * NS * nch, EC)
    p1 = _agg_call(g1, packed_f, norm_f, nch0, nch1, npad, h)
    g2 = _tcb_call(p1, b1.reshape(1, -1), W2, npad, blk)
    p2 = _agg_call(g2, packed_f, norm_f, nch0, nch1, npad, h)
    w3p = jnp.pad(W3, ((0, 0), (0, cp - c_out)))
    g3 = _tcb_call(p2, b2.reshape(1, -1), w3p, npad, blk)
    p3 = _agg_call(g3, packed_f, norm_f, nch0, nch1, npad, cp)
    b3p = jnp.pad(b3, (0, cp - c_out))
    o = _tcc_call(p3, b3p.reshape(1, -1), npad, blk, c_out)
    return o[:n, :c_out]


# speedup vs baseline: 1.1154x; 1.1154x over previous
import functools

import jax
import jax.numpy as jnp
from jax import lax
from jax.experimental import pallas as pl
from jax.experimental.pallas import tpu as pltpu
from jax.experimental.pallas import tpu_sc as plsc

NC = 2
NS = 16
LANES = 16
EC = 128


def _deg_body(nch, npad, dst_hbm, ew_hbm, out_hbm, dst_v, ew_v, acc_v):
    c = lax.axis_index("c")
    s = lax.axis_index("s")
    pltpu.sync_copy(dst_hbm.at[c, s], dst_v)
    pltpu.sync_copy(ew_hbm.at[c, s], ew_v)

    zero16 = jnp.zeros((LANES,), jnp.float32)

    def zr(i, _):
        acc_v[pl.ds(i * LANES, LANES)] = zero16
        return 0

    lax.fori_loop(0, npad // LANES, zr, 0)

    def chunk(ci, _):
        def grp(j, _):
            sl = pl.ds(j * LANES, LANES)
            plsc.addupdate_scatter(acc_v, [dst_v[ci, sl]], ew_v[ci, sl])
            return 0

        lax.fori_loop(0, EC // LANES, grp, 0)
        return 0

    lax.fori_loop(0, nch, chunk, 0)
    w = c * NS + s
    pltpu.sync_copy(acc_v, out_hbm.at[w])


def _deg_call(dst_r, ew_r, nch, npad):
    mesh = plsc.VectorSubcoreMesh(core_axis_name="c", subcore_axis_name="s")
    return pl.kernel(
        functools.partial(_deg_body, nch, npad),
        out_type=jax.ShapeDtypeStruct((NC * NS, npad), jnp.float32),
        mesh=mesh,
        compiler_params=pltpu.CompilerParams(needs_layout_passes=False),
        scratch_types=[
            pltpu.VMEM((nch, EC), jnp.int32),
            pltpu.VMEM((nch, EC), jnp.float32),
            pltpu.VMEM((npad,), jnp.float32),
        ],
    )(dst_r, ew_r)


def _norm_body(nch, npad, src_hbm, dst_hbm, ew_hbm, dis_hbm, out_hbm,
               src_v, dst_v, ew_v, dis_v, norm_v):
    c = lax.axis_index("c")
    s = lax.axis_index("s")
    pltpu.sync_copy(src_hbm.at[c, s], src_v)
    pltpu.sync_copy(dst_hbm.at[c, s], dst_v)
    pltpu.sync_copy(ew_hbm.at[c, s], ew_v)
    pltpu.sync_copy(dis_hbm, dis_v)

    def chunk(ci, _):
        def grp(j, _):
            sl = pl.ds(j * LANES, LANES)
            norm_v[ci, sl] = (plsc.load_gather(dis_v, [src_v[ci, sl]])
                              * ew_v[ci, sl]
                              * plsc.load_gather(dis_v, [dst_v[ci, sl]]))
            return 0

        lax.fori_loop(0, EC // LANES, grp, 0)
        return 0

    lax.fori_loop(0, nch, chunk, 0)
    pltpu.sync_copy(norm_v, out_hbm.at[c, s])


def _norm_call(src_r, dst_r, ew_r, dis, nch, npad):
    mesh = plsc.VectorSubcoreMesh(core_axis_name="c", subcore_axis_name="s")
    return pl.kernel(
        functools.partial(_norm_body, nch, npad),
        out_type=jax.ShapeDtypeStruct((NC, NS, nch, EC), jnp.float32),
        mesh=mesh,
        compiler_params=pltpu.CompilerParams(needs_layout_passes=False),
        scratch_types=[
            pltpu.VMEM((nch, EC), jnp.int32),
            pltpu.VMEM((nch, EC), jnp.int32),
            pltpu.VMEM((nch, EC), jnp.float32),
            pltpu.VMEM((npad,), jnp.float32),
            pltpu.VMEM((nch, EC), jnp.float32),
        ],
    )(src_r, dst_r, ew_r, dis)


ECA = 64
NBUF = 3


def _agg_body(nch0, nch1, npad, hd, tab_hbm, packed_hbm, norm_hbm, out_hbm,
              packed_v, norm_v, srcb, dstb, row0, row1, row2,
              acc_sh, sg0, sg1, sg2, ss0, ss1, ss2, sz):
    c = lax.axis_index("c")
    s = lax.axis_index("s")
    nch2 = jnp.where(c == 0, 2 * nch0, 2 * nch1)
    start = jnp.where(c == 0, s * nch0, NS * nch0 + s * nch1)
    rows = (row0, row1, row2)
    sgs = (sg0, sg1, sg2)
    sss = (ss0, ss1, ss2)
    pltpu.sync_copy(packed_hbm.at[pl.ds(start, max(nch0, nch1))], packed_v)
    pltpu.sync_copy(norm_hbm.at[pl.ds(start, max(nch0, nch1))], norm_v)

    zero16 = jnp.zeros((LANES,), jnp.float32)
    nj = hd // LANES

    def zrow(e, _):
        for j in range(nj):
            row2[e, pl.ds(j * LANES, LANES)] = zero16
        return 0

    lax.fori_loop(0, ECA, zrow, 0)

    rows_per_tile = npad // NS
    base = s * rows_per_tile
    nfull, rem = rows_per_tile // ECA, rows_per_tile % ECA
    zcopies = []
    for k in range(nfull):
        zcopies.append(pltpu.async_copy(
            row2, acc_sh.at[pl.ds(base + k * ECA, ECA)], sz))
    if rem:
        zcopies.append(pltpu.async_copy(
            row2.at[pl.ds(0, rem)],
            acc_sh.at[pl.ds(base + nfull * ECA, rem)], sz))

    def prep(prow, pcol, b):
        for q in range(ECA // LANES):
            sl = pl.ds(pcol * ECA + q * LANES, LANES)
            pv = packed_v[prow, sl]
            osl = pl.ds(q * LANES, LANES)
            srcb[b, osl] = lax.shift_right_logical(pv, 16)
            dstb[b, osl] = lax.bitwise_and(pv, 0xFFFF)
        pltpu.async_copy(tab_hbm.at[srcb.at[b]], rows[b], sgs[b])

    prep(0, 0, 0)
    prep(0, 1, 1)
    for zc in zcopies:
        zc.wait()
    plsc.subcore_barrier()

    def step(k, _):
        for u in range(6):
            ci = 6 * k + u
            b = u % 3
            r = 3 * k + u // 2
            col = u % 2
            pltpu.make_async_copy(tab_hbm.at[srcb.at[b]], rows[b],
                                  sgs[b]).wait()

            def sgrp(g, _):
                nv = norm_v[r, pl.ds(col * ECA + g * LANES, LANES)]
                for lane in range(LANES):
                    sc = jnp.full((LANES,), nv[lane], jnp.float32)
                    e = g * LANES + lane
                    for j in range(nj):
                        fsl = pl.ds(j * LANES, LANES)
                        rows[b][e, fsl] = rows[b][e, fsl] * sc
                return 0

            lax.fori_loop(0, ECA // LANES, sgrp, 0)
            pltpu.async_copy(rows[b], acc_sh.at[dstb.at[b]], sss[b],
                             add=True)

            bp = (b + 2) % 3

            @pl.when(ci >= 1)
            def _():
                pltpu.make_async_copy(rows[bp], acc_sh.at[dstb.at[bp]],
                                      sss[bp]).wait()

            @pl.when(ci + 2 < nch2)
            def _():
                prep(3 * k + (u + 2) // 2, (u + 2) % 2, bp)
        return 0

    lax.fori_loop(0, nch2 // 6, step, 0)
    pltpu.make_async_copy(rows[2], acc_sh.at[dstb.at[2]], sss[2]).wait()
    plsc.subcore_barrier()

    ocopies = []
    for k in range(nfull):
        ocopies.append(pltpu.async_copy(
            acc_sh.at[pl.ds(base + k * ECA, ECA)],
            out_hbm.at[c, pl.ds(base + k * ECA, ECA)], sz))
    if rem:
        ocopies.append(pltpu.async_copy(
            acc_sh.at[pl.ds(base + nfull * ECA, rem)],
            out_hbm.at[c, pl.ds(base + nfull * ECA, rem)], sz))
    for oc in ocopies:
        oc.wait()


def _agg_call(tab, packed_r, norm_r, nch0, nch1, npad, hd):
    mesh = plsc.VectorSubcoreMesh(core_axis_name="c", subcore_axis_name="s")
    nmax = max(nch0, nch1)
    return pl.kernel(
        functools.partial(_agg_body, nch0, nch1, npad, hd),
        out_type=jax.ShapeDtypeStruct((NC, npad, hd), jnp.float32),
        mesh=mesh,
        compiler_params=pltpu.CompilerParams(needs_layout_passes=False,
                                             use_tc_tiling_on_sc=False),
        scratch_types=[
            pltpu.VMEM((nmax, EC), jnp.int32),
            pltpu.VMEM((nmax, EC), jnp.float32),
            pltpu.VMEM((NBUF, ECA), jnp.int32),
            pltpu.VMEM((NBUF, ECA), jnp.int32),
            pltpu.VMEM((ECA, hd), jnp.float32),
            pltpu.VMEM((ECA, hd), jnp.float32),
            pltpu.VMEM((ECA, hd), jnp.float32),
            pltpu.MemorySpace.VMEM_SHARED((npad, hd), jnp.float32),
            pltpu.SemaphoreType.DMA,
            pltpu.SemaphoreType.DMA,
            pltpu.SemaphoreType.DMA,
            pltpu.SemaphoreType.DMA,
            pltpu.SemaphoreType.DMA,
            pltpu.SemaphoreType.DMA,
            pltpu.SemaphoreType.DMA,
        ],
    )(tab, packed_r, norm_r)


def _tca_body(x_ref, w_ref, degp_ref, g_ref, dis_ref):
    g_ref[...] = jnp.dot(x_ref[...], w_ref[...],
                         preferred_element_type=jnp.float32)

    @pl.when(pl.program_id(0) == 0)
    def _():
        deg = jnp.sum(degp_ref[...], axis=0)
        safe = jnp.where(deg > 0, deg, 1.0)
        dis_ref[...] = jnp.where(deg > 0, lax.rsqrt(safe), 0.0)


def _tca_call(x, w, degp3, n, blk):
    npad = degp3.shape[1] * degp3.shape[2]
    grid = (n // blk,)
    return pl.pallas_call(
        _tca_body,
        grid=grid,
        in_specs=[
            pl.BlockSpec((blk, x.shape[1]), lambda i: (i, 0)),
            pl.BlockSpec(w.shape, lambda i: (0, 0)),
            pl.BlockSpec(degp3.shape, lambda i: (0, 0, 0)),
        ],
        out_specs=[
            pl.BlockSpec((blk, w.shape[1]), lambda i: (i, 0)),
            pl.BlockSpec(degp3.shape[1:], lambda i: (0, 0)),
        ],
        out_shape=[
            jax.ShapeDtypeStruct((n, w.shape[1]), jnp.float32),
            jax.ShapeDtypeStruct(degp3.shape[1:], jnp.float32),
        ],
    )(x, w, degp3)


def _tcb_body(p_ref, b_ref, w_ref, g_ref):
    h = jax.nn.relu(p_ref[0] + p_ref[1] + b_ref[0, :])
    g_ref[...] = jnp.dot(h, w_ref[...], preferred_element_type=jnp.float32)


def _tcb_call(p, b2d, w, n, blk):
    hd_in = p.shape[2]
    return pl.pallas_call(
        _tcb_body,
        grid=(n // blk,),
        in_specs=[
            pl.BlockSpec((NC, blk, hd_in), lambda i: (0, i, 0)),
            pl.BlockSpec(b2d.shape, lambda i: (0, 0)),
            pl.BlockSpec(w.shape, lambda i: (0, 0)),
        ],
        out_specs=pl.BlockSpec((blk, w.shape[1]), lambda i: (i, 0)),
        out_shape=jax.ShapeDtypeStruct((n, w.shape[1]), jnp.float32),
    )(p, b2d, w)


def _tcc_body(nclass, p_ref, b_ref, o_ref):
    t = jax.nn.relu(p_ref[0] + p_ref[1] + b_ref[0, :])
    col = lax.broadcasted_iota(jnp.int32, t.shape, 1)
    tm = jnp.where(col < nclass, t, -jnp.inf)
    m = jnp.max(tm, axis=-1, keepdims=True)
    lse = m + jnp.log(jnp.sum(jnp.exp(tm - m), axis=-1, keepdims=True))
    o_ref[...] = t - lse


def _tcc_call(p, b2d, n, blk, nclass):
    cp = p.shape[2]
    return pl.pallas_call(
        functools.partial(_tcc_body, nclass),
        grid=(n // blk,),
        in_specs=[
            pl.BlockSpec((NC, blk, cp), lambda i: (0, i, 0)),
            pl.BlockSpec(b2d.shape, lambda i: (0, 0)),
        ],
        out_specs=pl.BlockSpec((blk, cp), lambda i: (i, 0)),
        out_shape=jax.ShapeDtypeStruct((n, cp), jnp.float32),
    )(p, b2d)


def kernel(x, edge_index, edge_attr, W1, b1, W2, b2, W3, b3):
    n, d = x.shape
    h = W1.shape[1]
    c_out = W3.shape[1]
    cp = 64
    e = edge_index.shape[1]
    e_f = e + n
    nch = -(-e_f // (NC * NS * EC))
    e_pad = NC * NS * nch * EC
    npad = -(-n // 128) * 128
    blk = npad // 8

    loop_idx = jnp.arange(n, dtype=edge_index.dtype)
    src_f = jnp.concatenate([edge_index[0], loop_idx,
                             jnp.zeros((e_pad - e_f,), edge_index.dtype)])
    dst_f = jnp.concatenate([edge_index[1], loop_idx,
                             jnp.zeros((e_pad - e_f,), edge_index.dtype)])
    ew_f = jnp.concatenate([edge_attr, jnp.ones((n,), jnp.float32),
                            jnp.zeros((e_pad - e_f,), jnp.float32)])
    src_r = src_f.reshape(NC, NS, nch, EC)
    dst_r = dst_f.reshape(NC, NS, nch, EC)
    ew_r = ew_f.reshape(NC, NS, nch, EC)

    xp = jnp.pad(x, ((0, npad - n), (0, 0)))
    degp = _deg_call(dst_r, ew_r, nch, npad)
    g1, dis2 = _tca_call(xp, W1, degp.reshape(NC * NS, npad // 128, 128),
                         npad, blk)
    dis = dis2.reshape(npad)
    norm_r = _norm_call(src_r, dst_r, ew_r, dis, nch, npad)

    packed_r = (src_f * 65536 + dst_f).reshape(NC, NS, nch, EC)

    nch0 = 2 * nch - (((2 * nch) * 13 // 32 + 3) // 6 * 6)
    nch1 = 2 * nch - nch0
    packed_f = packed_r.reshape(NC * NS * nch, EC)
    norm_f = norm_r.reshape(NC * NS * nch, EC)
    p1 = _agg_call(g1, packed_f, norm_f, nch0, nch1, npad, h)
    g2 = _tcb_call(p1, b1.reshape(1, -1), W2, npad, blk)
    p2 = _agg_call(g2, packed_f, norm_f, nch0, nch1, npad, h)
    w3p = jnp.pad(W3, ((0, 0), (0, cp - c_out)))
    g3 = _tcb_call(p2, b2.reshape(1, -1), w3p, npad, blk)
    p3 = _agg_call(g3, packed_f, norm_f, nch0, nch1, npad, cp)
    b3p = jnp.pad(b3, (0, cp - c_out))
    o = _tcc_call(p3, b3p.reshape(1, -1), npad, blk, c_out)
    return o[:n, :c_out]
